# baseline (speedup 1.0000x reference)
import functools

import jax
import jax.numpy as jnp
from jax import lax
from jax.experimental import pallas as pl
from jax.experimental.pallas import tpu as pltpu
from jax.experimental.pallas import tpu_sc as plsc

_NC = 2
_NS = 16
_NW = _NC * _NS
_EMB = 8
_LANES = 128
_CH = 128
_MAIN_ROWS = 999936
_MAIN_TILES = _MAIN_ROWS // _LANES


def _sc_gather_two(idx_u, idx_i, main_u, tail_u, main_i, tail_i):
    B = idx_u.shape[0]
    bpw = B // _NW
    nch = bpw // _CH
    nstr = nch * _EMB

    mesh = plsc.VectorSubcoreMesh(core_axis_name="c", subcore_axis_name="s")

    @functools.partial(
        pl.kernel,
        out_type=(jax.ShapeDtypeStruct((_EMB, B), jnp.float32),
                  jax.ShapeDtypeStruct((_EMB, B), jnp.float32)),
        mesh=mesh,
        scratch_types=[
            pltpu.VMEM((2, bpw), jnp.int32),
            pltpu.VMEM((2, nstr, _CH), jnp.int32),
            pltpu.VMEM((2, _EMB, bpw), jnp.float32),
            pltpu.VMEM((_EMB * _LANES,), jnp.float32),
            pltpu.VMEM((_EMB * _LANES,), jnp.float32),
            pltpu.SemaphoreType.DMA,
            pltpu.SemaphoreType.DMA,
        ],
        compiler_params=pltpu.CompilerParams(needs_layout_passes=False),
    )
    def gather(iu_h, ii_h, mu_h, tu_h, mi_h, ti_h, ou_h, oi_h,
               idx_v, sidx_v, col_v, tailu_v, taili_v, sem_u, sem_i):
        wid = lax.axis_index("s") * _NC + lax.axis_index("c")
        base = wid * bpw
        tails = [tailu_v, taili_v]

        pltpu.sync_copy(tu_h, tailu_v)
        pltpu.sync_copy(ti_h, taili_v)
        pltpu.sync_copy(iu_h.at[pl.ds(base, bpw)], idx_v.at[0])
        pltpu.sync_copy(ii_h.at[pl.ds(base, bpw)], idx_v.at[1])

        def fire(t, main_h, sem):
            for k in range(bpw // 16):
                v = idx_v[t, pl.ds(k * 16, 16)]
                tid = jnp.minimum(v >> 7, _MAIN_TILES - 1)
                b16 = tid * 1024 + (v & 127)
                j, kk = divmod(k, _CH // 16)
                for c in range(_EMB):
                    sidx_v[t, j * _EMB + c, pl.ds(kk * 16, 16)] = b16 + c * 128
            cps = []
            for j in range(nch):
                for c in range(_EMB):
                    cps.append(pltpu.async_copy(
                        main_h.at[sidx_v.at[t, j * _EMB + c]],
                        col_v.at[t, c, pl.ds(j * _CH, _CH)], sem))
            return cps

        def drain(t, cps, out_h):
            for cp in cps:
                cp.wait()

            def body(k, carry):
                v = idx_v[t, pl.ds(k * 16, 16)]
                m = v >= _MAIN_ROWS
                r = v & 127
                for c in range(_EMB):
                    tv = plsc.load_gather(tails[t], [r + c * 128])
                    cur = col_v[t, c, pl.ds(k * 16, 16)]
                    col_v[t, c, pl.ds(k * 16, 16)] = jnp.where(m, tv, cur)
                return carry
            lax.fori_loop(0, bpw // 16, body, 0)

            for c in range(_EMB):
                pltpu.sync_copy(col_v.at[t, c], out_h.at[c, pl.ds(base, bpw)])

        cps_u = fire(0, mu_h, sem_u)
        cps_i = fire(1, mi_h, sem_i)
        drain(0, cps_u, ou_h)
        drain(1, cps_i, oi_h)

    return gather(idx_u, idx_i, main_u, tail_u, main_i, tail_i)


def _tc_feat_partial(feat, Wf, bf, W1f, b1):
    B = feat.shape[0]
    BB = 2048

    def body(f_ref, wf_ref, bf_ref, w1f_ref, b1_ref, o_ref):
        dn11 = (((1,), (1,)), ((), ()))
        dn10 = (((1,), (0,)), ((), ()))
        fenc_t = lax.dot_general(wf_ref[...], f_ref[...], dn11,
                                 preferred_element_type=jnp.float32) + bf_ref[...]
        o_ref[...] = lax.dot_general(w1f_ref[...], fenc_t, dn10,
                                     preferred_element_type=jnp.float32) + b1_ref[...]

    return pl.pallas_call(
        body,
        grid=(B // BB,),
        in_specs=[
            pl.BlockSpec((BB, 128), lambda g: (g, 0)),
            pl.BlockSpec((_EMB, 128), lambda g: (0, 0)),
            pl.BlockSpec((_EMB, 1), lambda g: (0, 0)),
            pl.BlockSpec((64, _EMB), lambda g: (0, 0)),
            pl.BlockSpec((64, 1), lambda g: (0, 0)),
        ],
        out_specs=pl.BlockSpec((64, BB), lambda g: (0, g)),
        out_shape=jax.ShapeDtypeStruct((64, B), jnp.float32),
    )(feat, Wf, bf, W1f, b1)


def _tc_combine(partial_t, ucols, icols, W1u, W1i):
    B = partial_t.shape[1]
    BB = 4096

    def body(p_ref, u_ref, i_ref, w1u_ref, w1i_ref, o_ref):
        dn10 = (((1,), (0,)), ((), ()))
        h = (p_ref[...]
             + lax.dot_general(w1u_ref[...], u_ref[...], dn10,
                               preferred_element_type=jnp.float32)
             + lax.dot_general(w1i_ref[...], i_ref[...], dn10,
                               preferred_element_type=jnp.float32))
        o_ref[...] = jnp.maximum(h, 0.0)

    out_t = pl.pallas_call(
        body,
        grid=(B // BB,),
        in_specs=[
            pl.BlockSpec((64, BB), lambda g: (0, g)),
            pl.BlockSpec((_EMB, BB), lambda g: (0, g)),
            pl.BlockSpec((_EMB, BB), lambda g: (0, g)),
            pl.BlockSpec((64, _EMB), lambda g: (0, 0)),
            pl.BlockSpec((64, _EMB), lambda g: (0, 0)),
        ],
        out_specs=pl.BlockSpec((64, BB), lambda g: (0, g)),
        out_shape=jax.ShapeDtypeStruct((64, B), jnp.float32),
    )(partial_t, ucols, icols, W1u, W1i)
    return out_t.T


def _views(tab):
    main3 = tab[:_MAIN_ROWS].reshape(_MAIN_TILES, _LANES, _EMB).transpose(0, 2, 1)
    main3 = lax.optimization_barrier(main3)
    main = main3.reshape(-1)
    tail3 = jnp.pad(tab[_MAIN_ROWS:], ((0, _LANES - (tab.shape[0] - _MAIN_ROWS)),
                                       (0, 0))).reshape(1, _LANES, _EMB).transpose(0, 2, 1)
    tail3 = lax.optimization_barrier(tail3)
    tail = tail3.reshape(-1)
    return main, tail


def kernel(user_idx, item_idx, feat_vecs, user_table, item_table, Wf, bf, W1, b1):
    umain, utail = _views(user_table)
    imain, itail = _views(item_table)
    ucols, icols = _sc_gather_two(user_idx.astype(jnp.int32),
                                  item_idx.astype(jnp.int32),
                                  umain, utail, imain, itail)
    W1u = W1[:, 0:_EMB]
    W1i = W1[:, _EMB:2 * _EMB]
    W1f = W1[:, 2 * _EMB:3 * _EMB]
    partial_t = _tc_feat_partial(feat_vecs, Wf, bf.reshape(_EMB, 1),
                                 W1f, b1.reshape(64, 1))
    return _tc_combine(partial_t, ucols, icols, W1u, W1i)
